# Initial kernel scaffold; baseline (speedup 1.0000x reference)
import jax
import jax.numpy as jnp
from jax.experimental import pallas as pl


def kernel(x, edge_index, W_nb, b_nb, W_self, b_self, W_att, b_att):
    raise NotImplementedError("write your pallas kernel here")


# speedup vs baseline: 19.4828x; 19.4828x over previous
import functools

import jax
import jax.numpy as jnp
from jax import lax
from jax.experimental import pallas as pl
from jax.experimental.pallas import tpu as pltpu
from jax.experimental.pallas import tpu_sc as plsc

N_NODES = 10000
D_FEAT = 128
N_EDGES = 320000
HIDDEN = 16

NUM_WORKERS = 32
CHUNK = N_EDGES // NUM_WORKERS
LANES = 16


def _node_scalars_body(x_ref, wnb_ref, bnb_ref, wself_ref, bself_ref,
                       wa1_ref, wa2_ref, batt_ref, snb_ref, sself_ref):
    xv = x_ref[...]
    h1 = jnp.maximum(
        jnp.dot(xv, wnb_ref[...], preferred_element_type=jnp.float32)
        + bnb_ref[...], 0.0)
    h2 = jnp.maximum(
        jnp.dot(xv, wself_ref[...], preferred_element_type=jnp.float32)
        + bself_ref[...], 0.0)
    snb_ref[...] = (jnp.dot(h1, wa1_ref[...], preferred_element_type=jnp.float32)
                    + batt_ref[...])
    sself_ref[...] = jnp.dot(h2, wa2_ref[...], preferred_element_type=jnp.float32)


def _node_scalars(x, W_nb, b_nb, W_self, b_self, W_att, b_att):
    return pl.pallas_call(
        _node_scalars_body,
        out_shape=[
            jax.ShapeDtypeStruct((N_NODES, 1), jnp.float32),
            jax.ShapeDtypeStruct((N_NODES, 1), jnp.float32),
        ],
    )(x, W_nb, b_nb.reshape(1, HIDDEN), W_self, b_self.reshape(1, HIDDEN),
      W_att[:HIDDEN], W_att[HIDDEN:], b_att.reshape(1, 1))


def _edge_gate_body(snb_hbm, sself_hbm, row_hbm, col_hbm,
                    mask_hbm, psum_hbm,
                    snb_v, sself_v, row_v, col_v, mask_v, acc_v):
    wid = lax.axis_index("s") * 2 + lax.axis_index("c")
    base = wid * CHUNK
    pltpu.sync_copy(snb_hbm, snb_v)
    pltpu.sync_copy(sself_hbm, sself_v)
    pltpu.sync_copy(row_hbm.at[pl.ds(base, CHUNK)], row_v)
    pltpu.sync_copy(col_hbm.at[pl.ds(base, CHUNK)], col_v)

    def step(i, acc):
        off = i * LANES
        idx_r = row_v[pl.ds(off, LANES)]
        idx_c = col_v[pl.ds(off, LANES)]
        s1 = plsc.load_gather(snb_v, [idx_r])
        s2 = plsc.load_gather(sself_v, [idx_c])
        la = s1 + s2
        gate = 1.0 / (1.0 + jnp.exp(-la))
        m = jnp.minimum(jnp.maximum(gate * 1.01, 0.0), 1.0)
        mask_v[pl.ds(off, LANES)] = m
        return acc + m

    acc = lax.fori_loop(0, CHUNK // LANES, step,
                        jnp.zeros((LANES,), jnp.float32))
    acc_v[...] = acc
    pltpu.sync_copy(mask_v, mask_hbm.at[pl.ds(base, CHUNK)])
    pltpu.sync_copy(acc_v, psum_hbm.at[wid])


def _edge_gate(s_nb, s_self, row, col):
    mesh = plsc.VectorSubcoreMesh(core_axis_name="c", subcore_axis_name="s")
    fn = functools.partial(
        pl.kernel,
        mesh=mesh,
        compiler_params=pltpu.CompilerParams(needs_layout_passes=False),
        out_type=[
            jax.ShapeDtypeStruct((N_EDGES,), jnp.float32),
            jax.ShapeDtypeStruct((NUM_WORKERS, LANES), jnp.float32),
        ],
        scratch_types=[
            pltpu.VMEM((N_NODES,), jnp.float32),
            pltpu.VMEM((N_NODES,), jnp.float32),
            pltpu.VMEM((CHUNK,), jnp.int32),
            pltpu.VMEM((CHUNK,), jnp.int32),
            pltpu.VMEM((CHUNK,), jnp.float32),
            pltpu.VMEM((LANES,), jnp.float32),
        ],
    )(_edge_gate_body)
    return fn(s_nb, s_self, row, col)


def _sum_body(p_ref, out_ref):
    out_ref[...] = jnp.sum(p_ref[...]).reshape(1, 1)


def _sum_partials(partials):
    return pl.pallas_call(
        _sum_body,
        out_shape=jax.ShapeDtypeStruct((1, 1), jnp.float32),
    )(partials)


def kernel(x, edge_index, W_nb, b_nb, W_self, b_self, W_att, b_att):
    s_nb, s_self = _node_scalars(x, W_nb, b_nb, W_self, b_self, W_att, b_att)
    row = edge_index[0].astype(jnp.int32)
    col = edge_index[1].astype(jnp.int32)
    mask_flat, partials = _edge_gate(s_nb.reshape(N_NODES),
                                     s_self.reshape(N_NODES), row, col)
    mask_sum = _sum_partials(partials).reshape(())
    return mask_flat.reshape(N_EDGES, 1), mask_sum


# speedup vs baseline: 22.8277x; 1.1717x over previous
import functools

import jax
import jax.numpy as jnp
from jax import lax
from jax.experimental import pallas as pl
from jax.experimental.pallas import tpu as pltpu
from jax.experimental.pallas import tpu_sc as plsc

N_NODES = 10000
D_FEAT = 128
N_EDGES = 320000
HIDDEN = 16

NUM_WORKERS = 32
CHUNK = N_EDGES // NUM_WORKERS
LANES = 16
UNROLL = 5


def _node_scalars_body(x_ref, wnb_ref, bnb_ref, wself_ref, bself_ref,
                       wa1_ref, wa2_ref, batt_ref, snb_ref, sself_ref):
    xv = x_ref[...]
    h1 = jnp.maximum(
        jnp.dot(xv, wnb_ref[...], preferred_element_type=jnp.float32)
        + bnb_ref[...], 0.0)
    h2 = jnp.maximum(
        jnp.dot(xv, wself_ref[...], preferred_element_type=jnp.float32)
        + bself_ref[...], 0.0)
    snb_ref[...] = (jnp.dot(h1, wa1_ref[...], preferred_element_type=jnp.float32)
                    + batt_ref[...])
    sself_ref[...] = jnp.dot(h2, wa2_ref[...], preferred_element_type=jnp.float32)


def _node_scalars(x, W_nb, b_nb, W_self, b_self, W_att, b_att):
    return pl.pallas_call(
        _node_scalars_body,
        out_shape=[
            jax.ShapeDtypeStruct((N_NODES, 1), jnp.float32),
            jax.ShapeDtypeStruct((N_NODES, 1), jnp.float32),
        ],
    )(x, W_nb, b_nb.reshape(1, HIDDEN), W_self, b_self.reshape(1, HIDDEN),
      W_att[:HIDDEN], W_att[HIDDEN:], b_att.reshape(1, 1))


def _edge_gate_body(snb_hbm, sself_hbm, edge_hbm,
                    mask_hbm, psum_hbm,
                    snb_v, sself_v, row_v, col_v, mask_v, acc_v, sem):
    wid = lax.axis_index("s") * 2 + lax.axis_index("c")
    base = wid * CHUNK
    c1 = pltpu.async_copy(snb_hbm, snb_v, sem)
    c2 = pltpu.async_copy(sself_hbm, sself_v, sem)
    c3 = pltpu.async_copy(edge_hbm.at[pl.ds(base, CHUNK)], row_v, sem)
    c4 = pltpu.async_copy(edge_hbm.at[pl.ds(N_EDGES + base, CHUNK)], col_v, sem)
    c1.wait()
    c2.wait()
    c3.wait()
    c4.wait()

    def step(i, acc):
        for j in range(UNROLL):
            off = (i * UNROLL + j) * LANES
            idx_r = row_v[pl.ds(off, LANES)]
            idx_c = col_v[pl.ds(off, LANES)]
            s1 = plsc.load_gather(snb_v, [idx_r])
            s2 = plsc.load_gather(sself_v, [idx_c])
            la = s1 + s2
            m = jnp.minimum(1.01 / (1.0 + jnp.exp(-la)), 1.0)
            mask_v[pl.ds(off, LANES)] = m
            acc = acc + m
        return acc

    acc = lax.fori_loop(0, CHUNK // (LANES * UNROLL), step,
                        jnp.zeros((LANES,), jnp.float32))
    acc_v[...] = acc
    pltpu.sync_copy(mask_v, mask_hbm.at[pl.ds(base, CHUNK)])
    pltpu.sync_copy(acc_v, psum_hbm.at[wid])


def _edge_gate(s_nb, s_self, edge_index):
    mesh = plsc.VectorSubcoreMesh(core_axis_name="c", subcore_axis_name="s")
    fn = functools.partial(
        pl.kernel,
        mesh=mesh,
        compiler_params=pltpu.CompilerParams(needs_layout_passes=False),
        out_type=[
            jax.ShapeDtypeStruct((N_EDGES,), jnp.float32),
            jax.ShapeDtypeStruct((NUM_WORKERS, LANES), jnp.float32),
        ],
        scratch_types=[
            pltpu.VMEM((N_NODES,), jnp.float32),
            pltpu.VMEM((N_NODES,), jnp.float32),
            pltpu.VMEM((CHUNK,), jnp.int32),
            pltpu.VMEM((CHUNK,), jnp.int32),
            pltpu.VMEM((CHUNK,), jnp.float32),
            pltpu.VMEM((LANES,), jnp.float32),
            pltpu.SemaphoreType.DMA,
        ],
    )(_edge_gate_body)
    return fn(s_nb, s_self, edge_index)


def _sum_body(p_ref, out_ref):
    out_ref[...] = jnp.sum(p_ref[...]).reshape(1, 1)


def _sum_partials(partials):
    return pl.pallas_call(
        _sum_body,
        out_shape=jax.ShapeDtypeStruct((1, 1), jnp.float32),
    )(partials)


def kernel(x, edge_index, W_nb, b_nb, W_self, b_self, W_att, b_att):
    s_nb, s_self = _node_scalars(x, W_nb, b_nb, W_self, b_self, W_att, b_att)
    mask_flat, partials = _edge_gate(s_nb.reshape(N_NODES),
                                     s_self.reshape(N_NODES),
                                     edge_index.astype(jnp.int32).reshape(2 * N_EDGES))
    mask_sum = _sum_partials(partials).reshape(())
    return mask_flat.reshape(N_EDGES, 1), mask_sum


# speedup vs baseline: 27.3663x; 1.1988x over previous
import functools

import jax
import jax.numpy as jnp
from jax import lax
from jax.experimental import pallas as pl
from jax.experimental.pallas import tpu as pltpu
from jax.experimental.pallas import tpu_sc as plsc

N_NODES = 10000
D_FEAT = 128
N_EDGES = 320000
HIDDEN = 16

NUM_WORKERS = 32
CHUNK = N_EDGES // NUM_WORKERS
LANES = 16
UNROLL = 5


def _node_scalars_body(x_ref, wnb_ref, bnb_ref, wself_ref, bself_ref,
                       wa1_ref, wa2_ref, batt_ref, snb_ref, sself_ref):
    xv = x_ref[...]
    h1 = jnp.maximum(
        jnp.dot(xv, wnb_ref[...], preferred_element_type=jnp.float32)
        + bnb_ref[...], 0.0)
    h2 = jnp.maximum(
        jnp.dot(xv, wself_ref[...], preferred_element_type=jnp.float32)
        + bself_ref[...], 0.0)
    snb_ref[...] = (jnp.dot(h1, wa1_ref[...], preferred_element_type=jnp.float32)
                    + batt_ref[...])
    sself_ref[...] = jnp.dot(h2, wa2_ref[...], preferred_element_type=jnp.float32)


def _node_scalars(x, W_nb, b_nb, W_self, b_self, W_att, b_att):
    return pl.pallas_call(
        _node_scalars_body,
        out_shape=[
            jax.ShapeDtypeStruct((N_NODES, 1), jnp.float32),
            jax.ShapeDtypeStruct((N_NODES, 1), jnp.float32),
        ],
    )(x, W_nb, b_nb.reshape(1, HIDDEN), W_self, b_self.reshape(1, HIDDEN),
      W_att[:HIDDEN], W_att[HIDDEN:], b_att.reshape(1, 1))


def _edge_gate_body(snb_hbm, sself_hbm, edge_hbm,
                    mask_hbm, psum_hbm,
                    snb_v, sself_v, row_v, col_v, mask_v, acc_v, sem):
    wid = lax.axis_index("s") * 2 + lax.axis_index("c")
    base = wid * CHUNK
    c1 = pltpu.async_copy(snb_hbm, snb_v, sem)
    c2 = pltpu.async_copy(sself_hbm, sself_v, sem)
    c3 = pltpu.async_copy(edge_hbm.at[pl.ds(base, CHUNK)], row_v, sem)
    c4 = pltpu.async_copy(edge_hbm.at[pl.ds(N_EDGES + base, CHUNK)], col_v, sem)
    c1.wait()
    c2.wait()
    c3.wait()
    c4.wait()

    @plsc.parallel_loop(0, CHUNK, LANES, unroll=UNROLL,
                        carry=jnp.zeros((LANES,), jnp.float32))
    def acc(off, acc_in):
        idx_r = row_v[pl.ds(off, LANES)]
        idx_c = col_v[pl.ds(off, LANES)]
        s1 = plsc.load_gather(snb_v, [idx_r])
        s2 = plsc.load_gather(sself_v, [idx_c])
        la = s1 + s2
        m = jnp.minimum(1.01 / (1.0 + jnp.exp(-la)), 1.0)
        mask_v[pl.ds(off, LANES)] = m
        return acc_in + m
    acc_v[...] = acc
    pltpu.sync_copy(mask_v, mask_hbm.at[pl.ds(base, CHUNK)])
    pltpu.sync_copy(acc_v, psum_hbm.at[wid])


def _edge_gate(s_nb, s_self, edge_index):
    mesh = plsc.VectorSubcoreMesh(core_axis_name="c", subcore_axis_name="s")
    fn = functools.partial(
        pl.kernel,
        mesh=mesh,
        compiler_params=pltpu.CompilerParams(needs_layout_passes=False),
        out_type=[
            jax.ShapeDtypeStruct((N_EDGES,), jnp.float32),
            jax.ShapeDtypeStruct((NUM_WORKERS, LANES), jnp.float32),
        ],
        scratch_types=[
            pltpu.VMEM((N_NODES,), jnp.float32),
            pltpu.VMEM((N_NODES,), jnp.float32),
            pltpu.VMEM((CHUNK,), jnp.int32),
            pltpu.VMEM((CHUNK,), jnp.int32),
            pltpu.VMEM((CHUNK,), jnp.float32),
            pltpu.VMEM((LANES,), jnp.float32),
            pltpu.SemaphoreType.DMA,
        ],
    )(_edge_gate_body)
    return fn(s_nb, s_self, edge_index)


def _sum_body(p_ref, out_ref):
    out_ref[...] = jnp.sum(p_ref[...]).reshape(1, 1)


def _sum_partials(partials):
    return pl.pallas_call(
        _sum_body,
        out_shape=jax.ShapeDtypeStruct((1, 1), jnp.float32),
    )(partials)


def kernel(x, edge_index, W_nb, b_nb, W_self, b_self, W_att, b_att):
    s_nb, s_self = _node_scalars(x, W_nb, b_nb, W_self, b_self, W_att, b_att)
    mask_flat, partials = _edge_gate(s_nb.reshape(N_NODES),
                                     s_self.reshape(N_NODES),
                                     edge_index.astype(jnp.int32).reshape(2 * N_EDGES))
    mask_sum = _sum_partials(partials).reshape(())
    return mask_flat.reshape(N_EDGES, 1), mask_sum


# speedup vs baseline: 27.5774x; 1.0077x over previous
import functools

import jax
import jax.numpy as jnp
from jax import lax
from jax.experimental import pallas as pl
from jax.experimental.pallas import tpu as pltpu
from jax.experimental.pallas import tpu_sc as plsc

N_NODES = 10000
D_FEAT = 128
N_EDGES = 320000
HIDDEN = 16

NUM_WORKERS = 32
CHUNK = N_EDGES // NUM_WORKERS
LANES = 16
UNROLL = 5

TAB_ROWS = 80
BLK = 128
ROWS_PER_BLK = 8
GRID = TAB_ROWS // ROWS_PER_BLK


def _node_scalars_body(x_ref, wnb_ref, bnb_ref, wself_ref, bself_ref,
                       watt_ref, batt_ref, snb_ref, sself_ref):
    xv = x_ref[...]
    h1 = jnp.maximum(
        jnp.dot(xv, wnb_ref[...], preferred_element_type=jnp.float32)
        + bnb_ref[...], 0.0)
    h2 = jnp.maximum(
        jnp.dot(xv, wself_ref[...], preferred_element_type=jnp.float32)
        + bself_ref[...], 0.0)
    wa1 = watt_ref[0:HIDDEN, :]
    wa2 = watt_ref[HIDDEN:, :]

    def rows(h, wa):
        return jnp.concatenate([
            lax.dot_general(wa, h[r * BLK:(r + 1) * BLK, :],
                            (((0,), (1,)), ((), ())),
                            preferred_element_type=jnp.float32)
            for r in range(ROWS_PER_BLK)
        ], axis=0)

    snb_ref[...] = rows(h1, wa1) + batt_ref[0]
    sself_ref[...] = rows(h2, wa2)


def _node_scalars(x, W_nb, b_nb, W_self, b_self, W_att, b_att):
    return pl.pallas_call(
        _node_scalars_body,
        grid=(GRID,),
        in_specs=[
            pl.BlockSpec((ROWS_PER_BLK * BLK, D_FEAT), lambda i: (i, 0)),
            pl.BlockSpec((D_FEAT, HIDDEN), lambda i: (0, 0)),
            pl.BlockSpec((HIDDEN,), lambda i: (0,)),
            pl.BlockSpec((D_FEAT, HIDDEN), lambda i: (0, 0)),
            pl.BlockSpec((HIDDEN,), lambda i: (0,)),
            pl.BlockSpec((2 * HIDDEN, 1), lambda i: (0, 0)),
            pl.BlockSpec((1,), lambda i: (0,)),
        ],
        out_specs=[
            pl.BlockSpec((ROWS_PER_BLK, BLK), lambda i: (i, 0)),
            pl.BlockSpec((ROWS_PER_BLK, BLK), lambda i: (i, 0)),
        ],
        out_shape=[
            jax.ShapeDtypeStruct((TAB_ROWS, BLK), jnp.float32),
            jax.ShapeDtypeStruct((TAB_ROWS, BLK), jnp.float32),
        ],
    )(x, W_nb, b_nb, W_self, b_self, W_att, b_att)


def _edge_gate_body(snb_hbm, sself_hbm, row_hbm, col_hbm,
                    mask_hbm, psum_hbm,
                    snb_v, sself_v, row_v, col_v, mask_v, acc_v, sem):
    wid = lax.axis_index("s") * 2 + lax.axis_index("c")
    base = wid * CHUNK
    c1 = pltpu.async_copy(snb_hbm, snb_v, sem)
    c2 = pltpu.async_copy(sself_hbm, sself_v, sem)
    c3 = pltpu.async_copy(row_hbm.at[pl.ds(base, CHUNK)], row_v, sem)
    c4 = pltpu.async_copy(col_hbm.at[pl.ds(base, CHUNK)], col_v, sem)
    c1.wait()
    c2.wait()
    c3.wait()
    c4.wait()

    @plsc.parallel_loop(0, CHUNK, LANES, unroll=UNROLL,
                        carry=jnp.zeros((LANES,), jnp.float32))
    def acc(off, acc_in):
        idx_r = row_v[pl.ds(off, LANES)]
        idx_c = col_v[pl.ds(off, LANES)]
        s1 = plsc.load_gather(snb_v, [idx_r >> 7, idx_r & 127])
        s2 = plsc.load_gather(sself_v, [idx_c >> 7, idx_c & 127])
        la = s1 + s2
        m = jnp.minimum(1.01 / (1.0 + jnp.exp(-la)), 1.0)
        mask_v[pl.ds(off, LANES)] = m
        return acc_in + m

    acc_v[...] = acc
    pltpu.sync_copy(mask_v, mask_hbm.at[pl.ds(base, CHUNK)])
    pltpu.sync_copy(acc_v, psum_hbm.at[wid])


def _edge_gate(s_nb, s_self, row, col):
    mesh = plsc.VectorSubcoreMesh(core_axis_name="c", subcore_axis_name="s")
    fn = functools.partial(
        pl.kernel,
        mesh=mesh,
        compiler_params=pltpu.CompilerParams(needs_layout_passes=False),
        out_type=[
            jax.ShapeDtypeStruct((N_EDGES,), jnp.float32),
            jax.ShapeDtypeStruct((NUM_WORKERS, LANES), jnp.float32),
        ],
        scratch_types=[
            pltpu.VMEM((TAB_ROWS, BLK), jnp.float32),
            pltpu.VMEM((TAB_ROWS, BLK), jnp.float32),
            pltpu.VMEM((CHUNK,), jnp.int32),
            pltpu.VMEM((CHUNK,), jnp.int32),
            pltpu.VMEM((CHUNK,), jnp.float32),
            pltpu.VMEM((LANES,), jnp.float32),
            pltpu.SemaphoreType.DMA,
        ],
    )(_edge_gate_body)
    return fn(s_nb, s_self, row, col)


def _sum_body(p_ref, out_ref):
    out_ref[...] = jnp.sum(p_ref[...]).reshape(1, 1)


def _sum_partials(partials):
    return pl.pallas_call(
        _sum_body,
        out_shape=jax.ShapeDtypeStruct((1, 1), jnp.float32),
    )(partials)


def kernel(x, edge_index, W_nb, b_nb, W_self, b_self, W_att, b_att):
    s_nb, s_self = _node_scalars(x, W_nb, b_nb, W_self, b_self, W_att, b_att)
    row = edge_index[0].astype(jnp.int32)
    col = edge_index[1].astype(jnp.int32)
    mask_flat, partials = _edge_gate(s_nb, s_self, row, col)
    mask_sum = _sum_partials(partials).reshape(())
    return mask_flat.reshape(N_EDGES, 1), mask_sum


# speedup vs baseline: 36.5076x; 1.3238x over previous
import functools

import jax
import jax.numpy as jnp
from jax import lax
from jax.experimental import pallas as pl
from jax.experimental.pallas import tpu as pltpu
from jax.experimental.pallas import tpu_sc as plsc

N_NODES = 10000
D_FEAT = 128
N_EDGES = 320000
HIDDEN = 16

NUM_WORKERS = 32
CHUNK = N_EDGES // NUM_WORKERS
LANES = 16
UNROLL = 5

TAB_ROWS = 80
BLK = 128
ROWS_PER_BLK = 8
GRID = TAB_ROWS // ROWS_PER_BLK


def _node_scalars_body(x_ref, wnb_ref, bnb_ref, wself_ref, bself_ref,
                       watt_ref, batt_ref, snb_ref, sself_ref):
    xv = x_ref[...]
    h1 = jnp.maximum(
        jnp.dot(xv, wnb_ref[...], preferred_element_type=jnp.float32)
        + bnb_ref[...], 0.0)
    h2 = jnp.maximum(
        jnp.dot(xv, wself_ref[...], preferred_element_type=jnp.float32)
        + bself_ref[...], 0.0)
    wa1 = watt_ref[0:HIDDEN, :]
    wa2 = watt_ref[HIDDEN:, :]

    def rows(h, wa):
        return jnp.concatenate([
            lax.dot_general(wa, h[r * BLK:(r + 1) * BLK, :],
                            (((0,), (1,)), ((), ())),
                            preferred_element_type=jnp.float32)
            for r in range(ROWS_PER_BLK)
        ], axis=0)

    snb_ref[...] = rows(h1, wa1) + batt_ref[0]
    sself_ref[...] = rows(h2, wa2)


def _node_scalars(x, W_nb, b_nb, W_self, b_self, W_att, b_att):
    return pl.pallas_call(
        _node_scalars_body,
        grid=(GRID,),
        in_specs=[
            pl.BlockSpec((ROWS_PER_BLK * BLK, D_FEAT), lambda i: (i, 0)),
            pl.BlockSpec((D_FEAT, HIDDEN), lambda i: (0, 0)),
            pl.BlockSpec((HIDDEN,), lambda i: (0,)),
            pl.BlockSpec((D_FEAT, HIDDEN), lambda i: (0, 0)),
            pl.BlockSpec((HIDDEN,), lambda i: (0,)),
            pl.BlockSpec((2 * HIDDEN, 1), lambda i: (0, 0)),
            pl.BlockSpec((1,), lambda i: (0,)),
        ],
        out_specs=[
            pl.BlockSpec((ROWS_PER_BLK, BLK), lambda i: (i, 0)),
            pl.BlockSpec((ROWS_PER_BLK, BLK), lambda i: (i, 0)),
        ],
        out_shape=[
            jax.ShapeDtypeStruct((TAB_ROWS, BLK), jnp.float32),
            jax.ShapeDtypeStruct((TAB_ROWS, BLK), jnp.float32),
        ],
    )(x, W_nb, b_nb, W_self, b_self, W_att, b_att)


ETILE = 128
N_ETILES = N_EDGES // ETILE
TPW = N_ETILES // NUM_WORKERS
MAIN = TPW * ETILE
TAIL_T0 = TPW * NUM_WORKERS
N_TAIL = N_ETILES - TAIL_T0


def _edge_gate_body(snb_hbm, sself_hbm, edge_hbm,
                    mask_hbm, psum_hbm,
                    snb_v, sself_v, e_v, et_v, mask_v, mt_v, acc_v, sem):
    wid = lax.axis_index("s") * 2 + lax.axis_index("c")
    base = wid * MAIN
    c1 = pltpu.async_copy(snb_hbm, snb_v, sem)
    c2 = pltpu.async_copy(sself_hbm, sself_v, sem)
    c3 = pltpu.async_copy(edge_hbm.at[:, pl.ds(base, MAIN)], e_v, sem)
    c4 = pltpu.async_copy(
        edge_hbm.at[:, pl.ds((TAIL_T0 + wid % N_TAIL) * ETILE, ETILE)],
        et_v, sem)
    c1.wait()
    c2.wait()
    c3.wait()
    c4.wait()

    def gate(idx_r, idx_c):
        s1 = plsc.load_gather(snb_v, [idx_r >> 7, idx_r & 127])
        s2 = plsc.load_gather(sself_v, [idx_c >> 7, idx_c & 127])
        la = s1 + s2
        return jnp.minimum(1.01 / (1.0 + jnp.exp(-la)), 1.0)

    @plsc.parallel_loop(0, MAIN, LANES, unroll=UNROLL,
                        carry=jnp.zeros((LANES,), jnp.float32))
    def acc(off, acc_in):
        m = gate(e_v[0, pl.ds(off, LANES)], e_v[1, pl.ds(off, LANES)])
        mask_v[pl.ds(off, LANES)] = m
        return acc_in + m

    pltpu.sync_copy(mask_v, mask_hbm.at[pl.ds(base, MAIN)])

    @pl.when(wid < N_TAIL)
    def _tail():
        @plsc.parallel_loop(0, ETILE, LANES, unroll=ETILE // LANES,
                            carry=acc)
        def acc2(off, acc_in):
            m = gate(et_v[0, pl.ds(off, LANES)], et_v[1, pl.ds(off, LANES)])
            mt_v[pl.ds(off, LANES)] = m
            return acc_in + m

        acc_v[...] = acc2
        pltpu.sync_copy(
            mt_v, mask_hbm.at[pl.ds((TAIL_T0 + wid) * ETILE, ETILE)])

    @pl.when(wid >= N_TAIL)
    def _no_tail():
        acc_v[...] = acc

    pltpu.sync_copy(acc_v, psum_hbm.at[wid])


def _edge_gate(s_nb, s_self, edge_index):
    mesh = plsc.VectorSubcoreMesh(core_axis_name="c", subcore_axis_name="s")
    fn = functools.partial(
        pl.kernel,
        mesh=mesh,
        compiler_params=pltpu.CompilerParams(needs_layout_passes=False),
        out_type=[
            jax.ShapeDtypeStruct((N_EDGES,), jnp.float32),
            jax.ShapeDtypeStruct((NUM_WORKERS, LANES), jnp.float32),
        ],
        scratch_types=[
            pltpu.VMEM((TAB_ROWS, BLK), jnp.float32),
            pltpu.VMEM((TAB_ROWS, BLK), jnp.float32),
            pltpu.VMEM((2, MAIN), jnp.int32),
            pltpu.VMEM((2, ETILE), jnp.int32),
            pltpu.VMEM((MAIN,), jnp.float32),
            pltpu.VMEM((ETILE,), jnp.float32),
            pltpu.VMEM((LANES,), jnp.float32),
            pltpu.SemaphoreType.DMA,
        ],
    )(_edge_gate_body)
    return fn(s_nb, s_self, edge_index)


def _sum_body(p_ref, out_ref):
    out_ref[...] = jnp.sum(p_ref[...]).reshape(1, 1)


def _sum_partials(partials):
    return pl.pallas_call(
        _sum_body,
        out_shape=jax.ShapeDtypeStruct((1, 1), jnp.float32),
    )(partials)


def kernel(x, edge_index, W_nb, b_nb, W_self, b_self, W_att, b_att):
    s_nb, s_self = _node_scalars(x, W_nb, b_nb, W_self, b_self, W_att, b_att)
    mask_flat, partials = _edge_gate(s_nb, s_self,
                                     edge_index.astype(jnp.int32))
    mask_sum = _sum_partials(partials).reshape(())
    return mask_flat.reshape(N_EDGES, 1), mask_sum


# speedup vs baseline: 36.7462x; 1.0065x over previous
import functools

import jax
import jax.numpy as jnp
from jax import lax
from jax.experimental import pallas as pl
from jax.experimental.pallas import tpu as pltpu
from jax.experimental.pallas import tpu_sc as plsc

N_NODES = 10000
D_FEAT = 128
N_EDGES = 320000
HIDDEN = 16

NUM_WORKERS = 32
CHUNK = N_EDGES // NUM_WORKERS
LANES = 16
UNROLL = 5

TAB_ROWS = 80
BLK = 128
ROWS_PER_BLK = 8
GRID = TAB_ROWS // ROWS_PER_BLK


def _node_scalars_body(x_ref, wnb_ref, bnb_ref, wself_ref, bself_ref,
                       watt_ref, batt_ref, snb_ref, sself_ref):
    xv = x_ref[...]
    h1 = jnp.maximum(
        jnp.dot(xv, wnb_ref[...], preferred_element_type=jnp.float32)
        + bnb_ref[...], 0.0)
    h2 = jnp.maximum(
        jnp.dot(xv, wself_ref[...], preferred_element_type=jnp.float32)
        + bself_ref[...], 0.0)
    wa1 = watt_ref[0:HIDDEN, :]
    wa2 = watt_ref[HIDDEN:, :]

    def rows(h, wa):
        s = lax.dot_general(wa, h, (((0,), (1,)), ((), ())),
                            preferred_element_type=jnp.float32)
        return s.reshape(ROWS_PER_BLK, BLK)

    snb_ref[...] = rows(h1, wa1) + batt_ref[0]
    sself_ref[...] = rows(h2, wa2)


def _node_scalars(x, W_nb, b_nb, W_self, b_self, W_att, b_att):
    return pl.pallas_call(
        _node_scalars_body,
        grid=(GRID,),
        in_specs=[
            pl.BlockSpec((ROWS_PER_BLK * BLK, D_FEAT), lambda i: (i, 0)),
            pl.BlockSpec((D_FEAT, HIDDEN), lambda i: (0, 0)),
            pl.BlockSpec((HIDDEN,), lambda i: (0,)),
            pl.BlockSpec((D_FEAT, HIDDEN), lambda i: (0, 0)),
            pl.BlockSpec((HIDDEN,), lambda i: (0,)),
            pl.BlockSpec((2 * HIDDEN, 1), lambda i: (0, 0)),
            pl.BlockSpec((1,), lambda i: (0,)),
        ],
        out_specs=[
            pl.BlockSpec((ROWS_PER_BLK, BLK), lambda i: (i, 0)),
            pl.BlockSpec((ROWS_PER_BLK, BLK), lambda i: (i, 0)),
        ],
        out_shape=[
            jax.ShapeDtypeStruct((TAB_ROWS, BLK), jnp.float32),
            jax.ShapeDtypeStruct((TAB_ROWS, BLK), jnp.float32),
        ],
    )(x, W_nb, b_nb, W_self, b_self, W_att, b_att)


ETILE = 128
N_ETILES = N_EDGES // ETILE
TPW = N_ETILES // NUM_WORKERS
MAIN = TPW * ETILE
TAIL_T0 = TPW * NUM_WORKERS
N_TAIL = N_ETILES - TAIL_T0


def _edge_gate_body(snb_hbm, sself_hbm, edge_hbm,
                    mask_hbm, psum_hbm,
                    snb_v, sself_v, e_v, et_v, mask_v, mt_v, acc_v, sem):
    wid = lax.axis_index("s") * 2 + lax.axis_index("c")
    base = wid * MAIN
    c1 = pltpu.async_copy(snb_hbm, snb_v, sem)
    c2 = pltpu.async_copy(sself_hbm, sself_v, sem)
    c3 = pltpu.async_copy(edge_hbm.at[:, pl.ds(base, MAIN)], e_v, sem)
    c4 = pltpu.async_copy(
        edge_hbm.at[:, pl.ds((TAIL_T0 + wid % N_TAIL) * ETILE, ETILE)],
        et_v, sem)
    c1.wait()
    c2.wait()
    c3.wait()
    c4.wait()

    def gate(idx_r, idx_c):
        s1 = plsc.load_gather(snb_v, [idx_r >> 7, idx_r & 127])
        s2 = plsc.load_gather(sself_v, [idx_c >> 7, idx_c & 127])
        la = s1 + s2
        return jnp.minimum(1.01 / (1.0 + jnp.exp(-la)), 1.0)

    @plsc.parallel_loop(0, MAIN, LANES, unroll=UNROLL,
                        carry=jnp.zeros((LANES,), jnp.float32))
    def acc(off, acc_in):
        m = gate(e_v[0, pl.ds(off, LANES)], e_v[1, pl.ds(off, LANES)])
        mask_v[pl.ds(off, LANES)] = m
        return acc_in + m

    pltpu.sync_copy(mask_v, mask_hbm.at[pl.ds(base, MAIN)])

    @pl.when(wid < N_TAIL)
    def _tail():
        @plsc.parallel_loop(0, ETILE, LANES, unroll=ETILE // LANES,
                            carry=acc)
        def acc2(off, acc_in):
            m = gate(et_v[0, pl.ds(off, LANES)], et_v[1, pl.ds(off, LANES)])
            mt_v[pl.ds(off, LANES)] = m
            return acc_in + m

        acc_v[...] = acc2
        pltpu.sync_copy(
            mt_v, mask_hbm.at[pl.ds((TAIL_T0 + wid) * ETILE, ETILE)])

    @pl.when(wid >= N_TAIL)
    def _no_tail():
        acc_v[...] = acc

    pltpu.sync_copy(acc_v, psum_hbm.at[wid])


def _edge_gate(s_nb, s_self, edge_index):
    mesh = plsc.VectorSubcoreMesh(core_axis_name="c", subcore_axis_name="s")
    fn = functools.partial(
        pl.kernel,
        mesh=mesh,
        compiler_params=pltpu.CompilerParams(needs_layout_passes=False),
        out_type=[
            jax.ShapeDtypeStruct((N_EDGES,), jnp.float32),
            jax.ShapeDtypeStruct((NUM_WORKERS, LANES), jnp.float32),
        ],
        scratch_types=[
            pltpu.VMEM((TAB_ROWS, BLK), jnp.float32),
            pltpu.VMEM((TAB_ROWS, BLK), jnp.float32),
            pltpu.VMEM((2, MAIN), jnp.int32),
            pltpu.VMEM((2, ETILE), jnp.int32),
            pltpu.VMEM((MAIN,), jnp.float32),
            pltpu.VMEM((ETILE,), jnp.float32),
            pltpu.VMEM((LANES,), jnp.float32),
            pltpu.SemaphoreType.DMA,
        ],
    )(_edge_gate_body)
    return fn(s_nb, s_self, edge_index)


def _sum_body(p_ref, out_ref):
    out_ref[...] = jnp.sum(p_ref[...]).reshape(1, 1)


def _sum_partials(partials):
    return pl.pallas_call(
        _sum_body,
        out_shape=jax.ShapeDtypeStruct((1, 1), jnp.float32),
    )(partials)


def kernel(x, edge_index, W_nb, b_nb, W_self, b_self, W_att, b_att):
    s_nb, s_self = _node_scalars(x, W_nb, b_nb, W_self, b_self, W_att, b_att)
    mask_flat, partials = _edge_gate(s_nb, s_self,
                                     edge_index.astype(jnp.int32))
    mask_sum = _sum_partials(partials).reshape(())
    return mask_flat.reshape(N_EDGES, 1), mask_sum
